# Initial kernel scaffold; baseline (speedup 1.0000x reference)
import jax
import jax.numpy as jnp
from jax.experimental import pallas as pl


def kernel(protein_x, protein_batch, ligand_1d, ligand_x, ligand_edge_index, ligand_batch, Wp, bp, W_ih, W_hh, b_ih, b_hh, Winit, binit, gat_W, att_src, att_dst, gat_b, sag_Wrel, sag_Wroot, sag_b, Wc1, bc1, Wc2, bc2):
    raise NotImplementedError("write your pallas kernel here")


# speedup vs baseline: 43.0203x; 43.0203x over previous
import functools

import jax
import jax.numpy as jnp
from jax import lax
from jax.experimental import pallas as pl
from jax.experimental.pallas import tpu as pltpu
from jax.experimental.pallas import tpu_sc as plsc

F32 = jnp.float32
I32 = jnp.int32

HIDDEN = 64
B_GRAPHS = 256
NP_NODES = 100000
NL_NODES = 10000
N_EDGES = 320000
HEADS = 4
D = 2 * HIDDEN
STEPS = 3


_PR = 1000
_PNB = NP_NODES // _PR


def _protein_body(x_ref, bt_ref, wp_ref, bp_ref, wih_ref, whh_ref, bias_ref,
                  out_ref, px_scr, bt_scr):
    i = pl.program_id(0)
    px = lax.dot_general(x_ref[...], wp_ref[...], (((1,), (1,)), ((), ())),
                         preferred_element_type=F32) + bp_ref[...]
    px_scr[pl.ds(i * _PR, _PR), :] = px
    bt_scr[i] = bt_ref[0]

    @pl.when(i == _PNB - 1)
    def _():
        iota_g = lax.broadcasted_iota(I32, (B_GRAPHS, _PR), 0)

        def attend(h):
            def blk(j, carry):
                den, num = carry
                pxb = px_scr[pl.ds(j * _PR, _PR), :]
                btb = bt_scr[j]
                mask = (iota_g == jnp.broadcast_to(btb, (B_GRAPHS, _PR))
                        ).astype(F32)
                et = lax.dot_general(h, pxb, (((1,), (1,)), ((), ())),
                                     preferred_element_type=F32)
                wt = jnp.exp(et) * mask
                den = den + jnp.sum(wt, axis=1, keepdims=True)
                num = num + lax.dot_general(wt, pxb, (((1,), (0,)), ((), ())),
                                            preferred_element_type=F32)
                return den, num

            den, num = lax.fori_loop(
                0, _PNB, blk,
                (jnp.zeros((B_GRAPHS, 1), F32),
                 jnp.zeros((B_GRAPHS, HIDDEN), F32)))
            return num / (den + 1e-16)

        h = jnp.zeros((B_GRAPHS, HIDDEN), F32)
        c = jnp.zeros((B_GRAPHS, HIDDEN), F32)
        qs = jnp.zeros((B_GRAPHS, 2 * HIDDEN), F32)
        for _ in range(STEPS):
            gates = (lax.dot_general(qs, wih_ref[...], (((1,), (1,)), ((), ())),
                                     preferred_element_type=F32)
                     + lax.dot_general(h, whh_ref[...], (((1,), (1,)), ((), ())),
                                       preferred_element_type=F32)
                     + bias_ref[...])
            ig = gates[:, 0:HIDDEN]
            fg = gates[:, HIDDEN:2 * HIDDEN]
            gg = gates[:, 2 * HIDDEN:3 * HIDDEN]
            og = gates[:, 3 * HIDDEN:4 * HIDDEN]
            c = jax.nn.sigmoid(fg) * c + jax.nn.sigmoid(ig) * jnp.tanh(gg)
            h = jax.nn.sigmoid(og) * jnp.tanh(c)
            r = attend(h)
            qs = jnp.concatenate([h, r], axis=1)
        out_ref[...] = qs


def _protein_set2set(protein_x, batch3, wp, bp2, wih, whh, bias2):
    return pl.pallas_call(
        _protein_body,
        grid=(_PNB,),
        in_specs=[
            pl.BlockSpec((_PR, 128), lambda i: (i, 0)),
            pl.BlockSpec((1, 1, _PR), lambda i: (i, 0, 0)),
            pl.BlockSpec((HIDDEN, 128), lambda i: (0, 0)),
            pl.BlockSpec((1, HIDDEN), lambda i: (0, 0)),
            pl.BlockSpec((4 * HIDDEN, 2 * HIDDEN), lambda i: (0, 0)),
            pl.BlockSpec((4 * HIDDEN, HIDDEN), lambda i: (0, 0)),
            pl.BlockSpec((1, 4 * HIDDEN), lambda i: (0, 0)),
        ],
        out_specs=pl.BlockSpec((B_GRAPHS, 2 * HIDDEN), lambda i: (0, 0)),
        out_shape=jax.ShapeDtypeStruct((B_GRAPHS, 2 * HIDDEN), F32),
        scratch_shapes=[
            pltpu.VMEM((NP_NODES, HIDDEN), F32),
            pltpu.VMEM((_PNB, 1, _PR), I32),
        ],
    )(protein_x, batch3, wp, bp2, wih, whh, bias2)


def _ligprep_body(lx_ref, winit_ref, binit_ref, gatw_ref, as_ref,
                  xh_ref, side_ref):
    lx0 = jnp.maximum(
        lax.dot_general(lx_ref[...], winit_ref[...], (((1,), (1,)), ((), ())),
                        preferred_element_type=F32) + binit_ref[...], 0.0)
    xh = lax.dot_general(lx0, gatw_ref[...], (((1,), (1,)), ((), ())),
                         preferred_element_type=F32)
    xh_ref[...] = xh
    side_ref[...] = lax.dot_general(xh, as_ref[...], (((1,), (0,)), ((), ())),
                                    preferred_element_type=F32)


def _ligand_prep(ligand_x, winit, binit2, gat_w, side_mat):
    return pl.pallas_call(
        _ligprep_body,
        out_shape=[
            jax.ShapeDtypeStruct((NL_NODES, D), F32),
            jax.ShapeDtypeStruct((NL_NODES, 16), F32),
        ],
    )(ligand_x, winit, binit2, gat_w, side_mat)


_NC = 2
_NS = 16
_EW = N_EDGES // (_NC * _NS)
_SUP = 2000
_NSUP = _EW // _SUP
_SCH = _SUP // 16
_ROWQ = 624
_ROWL = 640
_DROWS = 632
_DRQ = 40
_DRL = 32


def _elog_body(src_hbm, dst_hbm, asrc_hbm, adst_hbm, k_hbm, w_out,
               asrc_v, adst_v, k_v, src_v, dst_v, wstage_v):
    c = lax.axis_index("c")
    s = lax.axis_index("s")
    wid = c * _NS + s
    pltpu.sync_copy(asrc_hbm, asrc_v)
    pltpu.sync_copy(adst_hbm, adst_v)
    pltpu.sync_copy(k_hbm, k_v)
    iota16 = lax.iota(I32, 16)
    base_e = wid * _EW

    def sup(j, _):
        eb = base_e + j * _SUP
        pltpu.sync_copy(src_hbm.at[pl.ds(eb, _SUP)], src_v)
        pltpu.sync_copy(dst_hbm.at[pl.ds(eb, _SUP)], dst_v)

        def chunk(i, _):
            s16 = src_v[pl.ds(i * 16, 16)]
            d16 = dst_v[pl.ds(i * 16, 16)]
            s4 = s16 * HEADS
            d4 = d16 * HEADS
            for h in range(HEADS):
                off = jnp.full((16,), h, I32)
                a = (plsc.load_gather(asrc_v, [s4 + off])
                     + plsc.load_gather(adst_v, [d4 + off]))
                w_h = (jnp.exp(jnp.maximum(a, a * 0.2))
                       * k_v[pl.ds(h * 16, 16)])
                plsc.store_scatter(wstage_v, [iota16 * 4 + i * 64 + h], w_h)
            return 0

        lax.fori_loop(0, _SCH, chunk, 0)
        pltpu.sync_copy(wstage_v, w_out.at[pl.ds(eb * 4, _SUP * 4)])
        return 0

    lax.fori_loop(0, _NSUP, sup, 0)


def _edge_logits(src, dst, asrc_flat, adst_flat, k_flat):
    mesh = plsc.VectorSubcoreMesh(core_axis_name="c", subcore_axis_name="s",
                                  num_cores=_NC, num_subcores=_NS)
    fn = pl.kernel(
        _elog_body,
        out_type=jax.ShapeDtypeStruct((N_EDGES * HEADS,), F32),
        mesh=mesh,
        scratch_types=[
            pltpu.VMEM((NL_NODES * HEADS,), F32),
            pltpu.VMEM((NL_NODES * HEADS,), F32),
            pltpu.VMEM((HEADS * 16,), F32),
            pltpu.VMEM((_SUP,), I32),
            pltpu.VMEM((_SUP,), I32),
            pltpu.VMEM((_SUP * HEADS,), F32),
        ],
        compiler_params=pltpu.CompilerParams(needs_layout_passes=False),
    )
    return fn(src, dst, asrc_flat, adst_flat, k_flat)


def _gat_edge_body(src_hbm, dst_hbm, w_hbm, xh_hbm,
                   num_out, den_out,
                   src_v, dst_v, w_v, rows_v, wrows_v,
                   wbuf_v, znum_v, num_sp, den_sp, sem):
    c = lax.axis_index("c")
    s = lax.axis_index("s")
    z16 = jnp.zeros((16,), F32)

    for r in range(16):
        for j in range(8):
            znum_v[r, pl.ds(j * 16, 16)] = z16
            wbuf_v[r, pl.ds(j * 16, 16)] = z16
    row0 = s * _ROWQ
    nz = jnp.where(s == _NS - 1, _ROWL // 16, _ROWQ // 16)

    def zloop(k, _):
        pltpu.sync_copy(znum_v, num_sp.at[pl.ds(row0 + k * 16, 16)])
        return 0

    lax.fori_loop(0, nz, zloop, 0)
    drow0 = s * _DRQ
    nzd = jnp.where(s == _NS - 1, _DRL // 8, _DRQ // 8)

    def zdloop(k, _):
        pltpu.sync_copy(znum_v.at[pl.ds(0, 8)],
                        den_sp.at[pl.ds(drow0 + k * 8, 8)])
        return 0

    lax.fori_loop(0, nzd, zdloop, 0)
    plsc.subcore_barrier()

    iota16 = lax.iota(I32, 16)
    base_e = (c * _NS + s) * _EW

    def sup(j, _):
        eb = base_e + j * _SUP
        pltpu.sync_copy(src_hbm.at[pl.ds(eb, _SUP)], src_v)
        pltpu.sync_copy(dst_hbm.at[pl.ds(eb, _SUP)], dst_v)
        pltpu.sync_copy(w_hbm.at[pl.ds(eb * 4, _SUP * 4)], w_v)

        def chunk(i, _):
            s16 = src_v[pl.ds(i * 16, 16)]
            d16 = dst_v[pl.ds(i * 16, 16)]
            cp = pltpu.async_copy(xh_hbm.at[s16], rows_v, sem)
            wlist = [plsc.load_gather(w_v, [iota16 * 4 + i * 64 + h])
                     for h in range(HEADS)]
            colbase = lax.rem(d16, jnp.full((16,), 16, I32)) * 8
            for h in range(HEADS):
                plsc.store_scatter(wbuf_v, [iota16, colbase + h], wlist[h])
            cp.wait()
            for e in range(16):
                for h in range(HEADS):
                    wsc = wlist[h][e]
                    for jj in range(2):
                        col = (h * 2 + jj) * 16
                        wrows_v[e, pl.ds(col, 16)] = (
                            rows_v[e, pl.ds(col, 16)] * wsc)
            drow = lax.div(d16, jnp.full((16,), 16, I32))
            pltpu.sync_copy(wrows_v, num_sp.at[d16], add=True)
            pltpu.sync_copy(wbuf_v, den_sp.at[drow], add=True)
            for h in range(HEADS):
                plsc.store_scatter(wbuf_v, [iota16, colbase + h], z16)
            return 0

        lax.fori_loop(0, _SCH, chunk, 0)
        return 0

    lax.fori_loop(0, _NSUP, sup, 0)
    plsc.subcore_barrier()

    nw = jnp.where(s == _NS - 1, _ROWL // 16, _ROWQ // 16)

    def wloop(k, _):
        r = row0 + k * 16
        pltpu.sync_copy(num_sp.at[pl.ds(r, 16)],
                        num_out.at[c, pl.ds(r, 16)])
        return 0

    lax.fori_loop(0, nw, wloop, 0)
    nwd = jnp.where(s == _NS - 1, _DRL // 8, _DRQ // 8)

    def wdloop(k, _):
        r = drow0 + k * 8
        pltpu.sync_copy(den_sp.at[pl.ds(r, 8)],
                        den_out.at[c, pl.ds(r, 8)])
        return 0

    lax.fori_loop(0, nwd, wdloop, 0)


def _gat_edges(src, dst, w_flat, xh):
    mesh = plsc.VectorSubcoreMesh(core_axis_name="c", subcore_axis_name="s",
                                  num_cores=_NC, num_subcores=_NS)
    fn = pl.kernel(
        _gat_edge_body,
        out_type=[
            jax.ShapeDtypeStruct((_NC, NL_NODES, D), F32),
            jax.ShapeDtypeStruct((_NC, _DROWS, D), F32),
        ],
        mesh=mesh,
        scratch_types=[
            pltpu.VMEM((_SUP,), I32),
            pltpu.VMEM((_SUP,), I32),
            pltpu.VMEM((_SUP * HEADS,), F32),
            pltpu.VMEM((16, D), F32),
            pltpu.VMEM((16, D), F32),
            pltpu.VMEM((16, D), F32),
            pltpu.VMEM((16, D), F32),
            pltpu.VMEM_SHARED((NL_NODES, D), F32),
            pltpu.VMEM_SHARED((_DROWS, D), F32),
            pltpu.SemaphoreType.DMA,
        ],
        compiler_params=pltpu.CompilerParams(needs_layout_passes=False),
    )
    return fn(src, dst, w_flat, xh)


_DR = 1000
_DNB = NL_NODES // _DR


def _combine_body(num_ref, den_ref, xh_ref, side_ref, c4_ref,
                  r4_ref, gatb_ref, wrel_ref, wroot_ref, sagb_ref,
                  lx_ref, t_ref, root_ref):
    num = num_ref[0] + num_ref[1]
    den4 = den_ref[0][:, 0:HEADS] + den_ref[1][:, 0:HEADS]
    side = side_ref[...]
    st = side[:, 0:HEADS] + side[:, HEADS:2 * HEADS]
    wself4 = jnp.exp(jnp.maximum(st, st * 0.2) - c4_ref[...])
    wself = lax.dot_general(wself4, r4_ref[...], (((1,), (0,)), ((), ())),
                            preferred_element_type=F32)
    den = lax.dot_general(den4, r4_ref[...], (((1,), (0,)), ((), ())),
                          preferred_element_type=F32) + wself
    numer = num + wself * xh_ref[...]
    lx = jnp.maximum(numer / (den + 1e-16) + gatb_ref[...], 0.0)
    lx_ref[...] = lx
    t_ref[...] = lax.dot_general(
        wrel_ref[...], lx, (((1,), (1,)), ((), ())),
        preferred_element_type=F32).reshape(1, 1, _DR)
    root_ref[...] = (lax.dot_general(
        wroot_ref[...], lx, (((1,), (1,)), ((), ())),
        preferred_element_type=F32) + sagb_ref[...]).reshape(1, 1, _DR)


def _gat_combine(num2, den2, xh, side, c4, r4, gatb2, wrel, wroot,
                 sagb2):
    return pl.pallas_call(
        _combine_body,
        grid=(_DNB,),
        in_specs=[
            pl.BlockSpec((_NC, _DR, D), lambda i: (0, i, 0)),
            pl.BlockSpec((_NC, _DR, 8), lambda i: (0, i, 0)),
            pl.BlockSpec((_DR, D), lambda i: (i, 0)),
            pl.BlockSpec((_DR, 16), lambda i: (i, 0)),
            pl.BlockSpec((1, HEADS), lambda i: (0, 0)),
            pl.BlockSpec((HEADS, D), lambda i: (0, 0)),
            pl.BlockSpec((1, D), lambda i: (0, 0)),
            pl.BlockSpec((1, D), lambda i: (0, 0)),
            pl.BlockSpec((1, D), lambda i: (0, 0)),
            pl.BlockSpec((1, 1), lambda i: (0, 0)),
        ],
        out_specs=[
            pl.BlockSpec((_DR, D), lambda i: (i, 0)),
            pl.BlockSpec((1, 1, _DR), lambda i: (i, 0, 0)),
            pl.BlockSpec((1, 1, _DR), lambda i: (i, 0, 0)),
        ],
        out_shape=[
            jax.ShapeDtypeStruct((NL_NODES, D), F32),
            jax.ShapeDtypeStruct((_DNB, 1, _DR), F32),
            jax.ShapeDtypeStruct((_DNB, 1, _DR), F32),
        ],
    )(num2, den2, xh, side, c4, r4, gatb2, wrel, wroot, sagb2)


def _sagg_body(src_hbm, dst_hbm, t_hbm, out_hbm, src_v, dst_v, t_v, tbuf_v,
               zbuf_v, acc_sp):
    c = lax.axis_index("c")
    s = lax.axis_index("s")
    z16 = jnp.zeros((16,), F32)
    for r in range(16):
        for j in range(8):
            tbuf_v[r, pl.ds(j * 16, 16)] = z16
            if r < 8:
                zbuf_v[r, pl.ds(j * 16, 16)] = z16
    drow0 = s * _DRQ
    nz = jnp.where(s == _NS - 1, _DRL // 8, _DRQ // 8)

    def zloop(k, _):
        pltpu.sync_copy(zbuf_v, acc_sp.at[pl.ds(drow0 + k * 8, 8)])
        return 0

    lax.fori_loop(0, nz, zloop, 0)
    pltpu.sync_copy(t_hbm, t_v)
    base = (c * _NS + s) * _EW
    pltpu.sync_copy(src_hbm.at[pl.ds(base, _EW)], src_v)
    pltpu.sync_copy(dst_hbm.at[pl.ds(base, _EW)], dst_v)
    plsc.subcore_barrier()

    iota16 = lax.iota(I32, 16)

    def chunk(i, _):
        s16 = src_v[pl.ds(i * 16, 16)]
        d16 = dst_v[pl.ds(i * 16, 16)]
        tv = plsc.load_gather(t_v, [s16])
        colb = lax.rem(d16, jnp.full((16,), 16, I32)) * 8
        plsc.store_scatter(tbuf_v, [iota16, colb], tv)
        drow = lax.div(d16, jnp.full((16,), 16, I32))
        pltpu.sync_copy(tbuf_v, acc_sp.at[drow], add=True)
        plsc.store_scatter(tbuf_v, [iota16, colb], z16)
        return 0

    lax.fori_loop(0, _EW // 16, chunk, 0)
    plsc.subcore_barrier()

    def wloop(k, _):
        r = drow0 + k * 8
        pltpu.sync_copy(acc_sp.at[pl.ds(r, 8)], out_hbm.at[c, pl.ds(r, 8)])
        return 0

    lax.fori_loop(0, nz, wloop, 0)


def _sag_agg(src, dst, t_flat):
    mesh = plsc.VectorSubcoreMesh(core_axis_name="c", subcore_axis_name="s",
                                  num_cores=_NC, num_subcores=_NS)
    fn = pl.kernel(
        _sagg_body,
        out_type=jax.ShapeDtypeStruct((_NC, _DROWS, D), F32),
        mesh=mesh,
        scratch_types=[
            pltpu.VMEM((_EW,), I32),
            pltpu.VMEM((_EW,), I32),
            pltpu.VMEM((NL_NODES,), F32),
            pltpu.VMEM((16, D), F32),
            pltpu.VMEM((8, D), F32),
            pltpu.VMEM_SHARED((_DROWS, D), F32),
        ],
        compiler_params=pltpu.CompilerParams(needs_layout_passes=False),
    )
    return fn(src, dst, t_flat)


_FR = 1000
_FNB = NL_NODES // _FR


def _readout_body(sagg_ref, root_ref, lx_ref, bt_ref, gp_ref, wc1_ref,
                  bc1_ref, wc2_ref, bc2_ref, out_ref):
    iota_g = lax.broadcasted_iota(I32, (B_GRAPHS, _FR), 0)

    def blk(j, carry):
        den, g = carry
        sc = sagg_ref[0, j] + sagg_ref[1, j] + root_ref[j]
        btb = bt_ref[j]
        mask = (iota_g == jnp.broadcast_to(btb, (B_GRAPHS, _FR))).astype(F32)
        wt = jnp.broadcast_to(jnp.exp(sc), (B_GRAPHS, _FR)) * mask
        den = den + jnp.sum(wt, axis=1, keepdims=True)
        lxb = lx_ref[pl.ds(j * _FR, _FR), :]
        g = g + lax.dot_general(wt, lxb, (((1,), (0,)), ((), ())),
                                preferred_element_type=F32)
        return den, g

    den, g = lax.fori_loop(0, _FNB, blk,
                           (jnp.zeros((B_GRAPHS, 1), F32),
                            jnp.zeros((B_GRAPHS, D), F32)))
    gl = g / (den + 1e-16)
    gf = jnp.concatenate([gp_ref[...], gl], axis=1)
    hm = jnp.maximum(
        lax.dot_general(gf, wc1_ref[...], (((1,), (1,)), ((), ())),
                        preferred_element_type=F32) + bc1_ref[...], 0.0)
    out_ref[...] = lax.dot_general(hm, wc2_ref[...], (((1,), (1,)), ((), ())),
                                   preferred_element_type=F32) + bc2_ref[...]


def _readout(sagg, root_t, lx, batch2, gp, wc1, bc12, wc2, bc22):
    return pl.pallas_call(
        _readout_body,
        out_shape=jax.ShapeDtypeStruct((B_GRAPHS, 2), F32),
    )(sagg, root_t, lx, batch2, gp, wc1, bc12, wc2, bc22)


def kernel(protein_x, protein_batch, ligand_1d, ligand_x, ligand_edge_index,
           ligand_batch, Wp, bp, W_ih, W_hh, b_ih, b_hh, Winit, binit, gat_W,
           att_src, att_dst, gat_b, sag_Wrel, sag_Wroot, sag_b, Wc1, bc1,
           Wc2, bc2):
    del ligand_1d

    batch3 = protein_batch.astype(I32).reshape(_PNB, 1, _PR)
    gp = _protein_set2set(protein_x, batch3, Wp, bp.reshape(1, -1), W_ih,
                          W_hh, (b_ih + b_hh).reshape(1, -1))

    head_of_col = jnp.repeat(jnp.arange(HEADS), D // HEADS)
    side_mat = jnp.zeros((D, 16), F32)
    side_mat = side_mat.at[jnp.arange(D), head_of_col].set(
        att_src.reshape(-1))
    side_mat = side_mat.at[jnp.arange(D), HEADS + head_of_col].set(
        att_dst.reshape(-1))
    xh, side = _ligand_prep(ligand_x, Winit, binit.reshape(1, -1),
                            gat_W, side_mat)

    smax = (jnp.max(side[:, 0:HEADS], axis=0)
            + jnp.max(side[:, HEADS:2 * HEADS], axis=0))
    c4 = jnp.maximum(smax, 0.2 * smax)
    k_flat = jnp.repeat(jnp.exp(-c4), 16)

    src = ligand_edge_index[0].astype(I32)
    dst = ligand_edge_index[1].astype(I32)

    asrc_flat = side[:, 0:HEADS].reshape(-1)
    adst_flat = side[:, HEADS:2 * HEADS].reshape(-1)
    w_flat = _edge_logits(src, dst, asrc_flat, adst_flat, k_flat)
    num2, den2 = _gat_edges(src, dst, w_flat, xh)
    den8 = den2.reshape(_NC, _DROWS * 16, 8)[:, :NL_NODES, :]

    r4 = (head_of_col[None, :] == jnp.arange(HEADS)[:, None]).astype(F32)
    lx, t3, root3 = _gat_combine(num2, den8, xh, side,
                                 c4.reshape(1, HEADS), r4,
                                 gat_b.reshape(1, -1), sag_Wrel, sag_Wroot,
                                 sag_b.reshape(1, 1))

    saggw = _sag_agg(src, dst, t3.reshape(-1))
    sagg = saggw.reshape(_NC, _DROWS * 16, 8)[:, :NL_NODES, 0]

    sagg4 = sagg.reshape(_NC, _FNB, 1, _FR)
    batch3 = ligand_batch.astype(I32).reshape(_FNB, 1, _FR)
    return _readout(sagg4, root3, lx, batch3, gp, Wc1, bc1.reshape(1, -1),
                    Wc2, bc2.reshape(1, -1))
